# baseline (speedup 1.0000x reference)
import functools

import jax
import jax.numpy as jnp
from jax import lax
from jax.experimental import pallas as pl
from jax.experimental.pallas import tpu as pltpu
from jax.experimental.pallas import tpu_sc as plsc


def _nonneg(w):
    return jnp.where(w > 0, w + 1.0, jnp.exp(jnp.minimum(w, 0.0)))


def _emb_body(x3_ref, wq_ref, wk_ref, q_ref, k_ref, *, pack, d, kdeg):
    wq = _nonneg(wq_ref[...]) * (1.0 / d)
    wk = _nonneg(wk_ref[...]) * (1.0 / (d * kdeg))
    dn = (((1,), (1,)), ((), ()))
    qs = []
    ks = []
    for a in range(pack):
        xa = x3_ref[:, a, :]
        qs.append(lax.dot_general(
            xa, wq, dn, preferred_element_type=jnp.float32))
        ks.append(lax.dot_general(
            xa, wk, dn, preferred_element_type=jnp.float32))
    q_ref[...] = jnp.concatenate(qs, axis=1)
    k_ref[...] = jnp.concatenate(ks, axis=1)


def _embeddings(x3, wq, wk, kdeg, npad, pack, block_nodes):
    d = x3.shape[2]
    h = wq.shape[0]
    grid = npad // block_nodes
    bpk = block_nodes // pack
    return pl.pallas_call(
        functools.partial(_emb_body, pack=pack, d=d, kdeg=kdeg),
        grid=(grid,),
        in_specs=[
            pl.BlockSpec((bpk, pack, d), lambda i: (i, 0, 0)),
            pl.BlockSpec((h, d), lambda i: (0, 0)),
            pl.BlockSpec((h, d), lambda i: (0, 0)),
        ],
        out_specs=[
            pl.BlockSpec((bpk, 128), lambda i: (i, 0)),
            pl.BlockSpec((bpk, 128), lambda i: (i, 0)),
        ],
        out_shape=[
            jax.ShapeDtypeStruct((npad // pack, 128), jnp.float32),
            jax.ShapeDtypeStruct((npad // pack, 128), jnp.float32),
        ],
    )(x3, wq, wk)


def _sc_edge_sum(adj3, q_emb, k_s, *, npad, c_nodes, kdeg):
    h = q_emb.shape[1]
    n_tab = q_emb.shape[0]
    e = adj3.shape[0] * 128
    info = plsc.get_sparse_core_info()
    nc, ns = info.num_cores, info.num_subcores
    nw = nc * ns
    np_w = npad // nw
    nchunk = np_w // c_nodes
    assert nchunk % 2 == 0 and np_w % c_nodes == 0
    ec = c_nodes * kdeg
    eblk = ec // 128
    assert ec % 128 == 0
    eb_max = e - ec
    assert eb_max % kdeg == 0 and eb_max % 128 == 0
    mesh = plsc.VectorSubcoreMesh(core_axis_name="c", subcore_axis_name="s")

    @functools.partial(
        pl.kernel,
        mesh=mesh,
        out_type=jax.ShapeDtypeStruct((npad, h), jnp.float32),
        scratch_types=[
            pltpu.VMEM((eblk, 128), jnp.int32),
            pltpu.VMEM((eblk, 128), jnp.int32),
            pltpu.VMEM((eblk, 128), jnp.int32),
            pltpu.VMEM((eblk, 128), jnp.int32),
            pltpu.VMEM((ec, h), jnp.float32),
            pltpu.VMEM((ec, h), jnp.float32),
            pltpu.VMEM((ec, h), jnp.float32),
            pltpu.VMEM((ec, h), jnp.float32),
            pltpu.VMEM((c_nodes, h), jnp.float32),
            pltpu.VMEM((c_nodes, h), jnp.float32),
            pltpu.VMEM_SHARED((n_tab, h), jnp.float32),
            pltpu.VMEM_SHARED((n_tab, h), jnp.float32),
            pltpu.SemaphoreType.DMA,
            pltpu.SemaphoreType.DMA,
            pltpu.SemaphoreType.DMA,
            pltpu.SemaphoreType.DMA,
            pltpu.SemaphoreType.DMA,
            pltpu.SemaphoreType.DMA,
        ],
        compiler_params=pltpu.CompilerParams(use_tc_tiling_on_sc=False,
                                             skip_device_barrier=True),
    )
    def run(adj_hbm, q_hbm, k_hbm, out_hbm,
            di0, si0, di1, si1, qr0, kr0, qr1, kr1, ob0, ob1, qs, ks,
            semi0, semi1, semg0, semg1, semo0, semo1):
        wid = lax.axis_index("s") * nc + lax.axis_index("c")
        ebase0 = wid * (np_w * kdeg)
        nbase0 = wid * np_w

        def eb_of(cix):
            raw = ebase0 + cix * ec
            return jnp.minimum(raw, eb_max), raw

        def start_idx(cix, di, si, sem):
            ebc, _ = eb_of(cix)
            bs = ebc // 128
            pltpu.async_copy(adj_hbm.at[pl.ds(bs, eblk), 1], di, sem)
            pltpu.async_copy(adj_hbm.at[pl.ds(bs, eblk), 0], si, sem)

        def wait_idx(di, si, sem):
            pltpu.make_async_copy(adj_hbm.at[pl.ds(0, eblk), 1], di, sem).wait()
            pltpu.make_async_copy(adj_hbm.at[pl.ds(0, eblk), 0], si, sem).wait()

        def start_gather(di, si, qr, kr, sem):
            for j in range(eblk):
                pltpu.async_copy(qs.at[di.at[j]],
                                 qr.at[pl.ds(j * 128, 128)], sem)
                pltpu.async_copy(ks.at[si.at[j]],
                                 kr.at[pl.ds(j * 128, 128)], sem)

        def wait_gather(di, si, qr, kr, sem):
            for j in range(eblk):
                pltpu.make_async_copy(qs.at[di.at[j]],
                                      qr.at[pl.ds(j * 128, 128)], sem).wait()
                pltpu.make_async_copy(ks.at[si.at[j]],
                                      kr.at[pl.ds(j * 128, 128)], sem).wait()

        def compute(cix, qr, kr, ob):
            ebc, raw = eb_of(cix)
            delta = raw - ebc

            def node_body(nix):
                off = jnp.minimum(nix * kdeg + delta, ec - kdeg)
                acc = qr[off] * kr[off]
                for j in range(1, kdeg):
                    acc = acc + qr[off + j] * kr[off + j]
                ob[nix] = acc

            plsc.parallel_loop(0, c_nodes, 1, unroll=2)(node_body)

        def start_out(cix, ob, sem):
            pltpu.async_copy(
                ob, out_hbm.at[pl.ds(nbase0 + cix * c_nodes, c_nodes)], sem)

        def wait_out(ob, sem):
            pltpu.make_async_copy(
                ob, out_hbm.at[pl.ds(0, c_nodes)], sem).wait()

        start_idx(0, di0, si0, semi0)
        start_idx(1, di1, si1, semi1)
        sid = lax.axis_index("s")
        rows16 = n_tab // ns
        pltpu.sync_copy(q_hbm.at[pl.ds(sid * rows16, rows16)],
                        qs.at[pl.ds(sid * rows16, rows16)])
        pltpu.sync_copy(k_hbm.at[pl.ds(sid * rows16, rows16)],
                        ks.at[pl.ds(sid * rows16, rows16)])
        plsc.subcore_barrier()
        wait_idx(di0, si0, semi0)
        start_gather(di0, si0, qr0, kr0, semg0)

        def pair_body(t, carry):
            c0 = 2 * t
            c1 = c0 + 1
            wait_idx(di1, si1, semi1)
            start_gather(di1, si1, qr1, kr1, semg1)
            wait_gather(di0, si0, qr0, kr0, semg0)

            @pl.when(c0 + 2 < nchunk)
            def _():
                start_idx(c0 + 2, di0, si0, semi0)

            compute(c0, qr0, kr0, ob0)

            @pl.when(t > 0)
            def _():
                wait_out(ob0, semo0)

            start_out(c0, ob0, semo0)

            @pl.when(c0 + 2 < nchunk)
            def _():
                wait_idx(di0, si0, semi0)
                start_gather(di0, si0, qr0, kr0, semg0)

            wait_gather(di1, si1, qr1, kr1, semg1)

            @pl.when(c1 + 2 < nchunk)
            def _():
                start_idx(c1 + 2, di1, si1, semi1)

            compute(c1, qr1, kr1, ob1)

            @pl.when(t > 0)
            def _():
                wait_out(ob1, semo1)

            start_out(c1, ob1, semo1)
            return carry

        lax.fori_loop(0, nchunk // 2, pair_body, 0)
        wait_out(ob0, semo0)
        wait_out(ob1, semo1)

    return run(adj3, q_emb, k_s)


def _epi_body(q_ref, s_ref, we_ref, wv_ref, o_ref, *, h, pack):
    q = q_ref[...]
    we = _nonneg(we_ref[...])[0:1, :]
    s = we * (q * q) + s_ref[...]
    ri = lax.broadcasted_iota(jnp.int32, (128, 128), 0) // h
    ci = lax.broadcasted_iota(jnp.int32, (128, 128), 1) // h
    grp = (ri == ci).astype(jnp.float32)
    dn = (((1,), (0,)), ((), ()))
    norm = lax.dot_general(s, grp, dn, preferred_element_type=jnp.float32)
    attn = s / (norm + 1e-9)
    wv = _nonneg(wv_ref[...])
    dnt = (((1,), (1,)), ((), ()))
    for a in range(pack):
        o_ref[:, a, :] = lax.dot_general(
            attn[:, a * h:(a + 1) * h], wv, dnt,
            preferred_element_type=jnp.float32)


def _epilogue(q_pk, s_pk, we_tile, wv, n, npad, h, dout, pack, block_nodes):
    grid = npad // block_nodes
    bpk = block_nodes // pack
    return pl.pallas_call(
        functools.partial(_epi_body, h=h, pack=pack),
        grid=(grid,),
        in_specs=[
            pl.BlockSpec((bpk, 128), lambda i: (i, 0)),
            pl.BlockSpec((bpk, 128), lambda i: (i, 0)),
            pl.BlockSpec((8, 128), lambda i: (0, 0)),
            pl.BlockSpec((dout, h), lambda i: (0, 0)),
        ],
        out_specs=pl.BlockSpec((bpk, pack, dout), lambda i: (i, 0, 0)),
        out_shape=jax.ShapeDtypeStruct((n // pack, pack, dout), jnp.float32),
    )(q_pk, s_pk, we_tile, wv)


def kernel(adj_list, x, Wq, Wk, w_ego, Wv):
    n, d = x.shape
    e = adj_list.shape[1]
    h = Wq.shape[0]
    dout = Wv.shape[0]
    kdeg = e // n
    pack = 128 // h

    c_nodes = 32
    nw = 32
    npad = ((n + nw * c_nodes - 1) // (nw * c_nodes)) * (nw * c_nodes)

    adj3 = jnp.transpose(adj_list.reshape(2, e // 128, 128), (1, 0, 2))
    x3 = x.reshape(n // pack, pack, d)

    we_tile = jnp.tile(w_ego, (8, pack))

    q_pk, k_pk = _embeddings(x3, Wq, Wk, kdeg, npad, pack, block_nodes=2048)
    q_tab = q_pk.reshape(npad, h)
    k_tab = k_pk.reshape(npad, h)

    sum_local_pad = _sc_edge_sum(adj3, q_tab, k_tab,
                                 npad=npad, c_nodes=c_nodes, kdeg=kdeg)
    s_pk = sum_local_pad.reshape(npad // pack, 128)

    res3 = _epilogue(q_pk, s_pk, we_tile, Wv, n, npad, h, dout, pack,
                     block_nodes=2048)
    return res3.reshape(n, dout)


# speedup vs baseline: 1.0248x; 1.0248x over previous
import functools

import jax
import jax.numpy as jnp
from jax import lax
from jax.experimental import pallas as pl
from jax.experimental.pallas import tpu as pltpu
from jax.experimental.pallas import tpu_sc as plsc


def _nonneg(w):
    return jnp.where(w > 0, w + 1.0, jnp.exp(jnp.minimum(w, 0.0)))


def _emb_body(x3_ref, wq_ref, wk_ref, q_ref, k_ref, *, pack, d, kdeg):
    wq = _nonneg(wq_ref[...]) * (1.0 / d)
    wk = _nonneg(wk_ref[...]) * (1.0 / (d * kdeg))
    dn = (((1,), (1,)), ((), ()))
    qs = []
    ks = []
    for a in range(pack):
        xa = x3_ref[:, a, :]
        qs.append(lax.dot_general(
            xa, wq, dn, preferred_element_type=jnp.float32))
        ks.append(lax.dot_general(
            xa, wk, dn, preferred_element_type=jnp.float32))
    q_ref[...] = jnp.concatenate(qs, axis=1)
    k_ref[...] = jnp.concatenate(ks, axis=1)


def _embeddings(x3, wq, wk, kdeg, npad, pack, block_nodes):
    d = x3.shape[2]
    h = wq.shape[0]
    grid = npad // block_nodes
    bpk = block_nodes // pack
    return pl.pallas_call(
        functools.partial(_emb_body, pack=pack, d=d, kdeg=kdeg),
        grid=(grid,),
        in_specs=[
            pl.BlockSpec((bpk, pack, d), lambda i: (i, 0, 0)),
            pl.BlockSpec((h, d), lambda i: (0, 0)),
            pl.BlockSpec((h, d), lambda i: (0, 0)),
        ],
        out_specs=[
            pl.BlockSpec((bpk, 128), lambda i: (i, 0)),
            pl.BlockSpec((bpk, 128), lambda i: (i, 0)),
        ],
        out_shape=[
            jax.ShapeDtypeStruct((npad // pack, 128), jnp.float32),
            jax.ShapeDtypeStruct((npad // pack, 128), jnp.float32),
        ],
    )(x3, wq, wk)


def _sc_edge_sum(adj3, q_emb, k_s, *, npad, c_nodes, kdeg):
    h = q_emb.shape[1]
    n_tab = q_emb.shape[0]
    e = adj3.shape[0] * 128
    info = plsc.get_sparse_core_info()
    nc, ns = info.num_cores, info.num_subcores
    nw = nc * ns
    np_w = npad // nw
    nchunk = np_w // c_nodes
    assert nchunk % 2 == 0 and np_w % c_nodes == 0
    ec = c_nodes * kdeg
    eblk = ec // 128
    assert ec % 128 == 0
    eb_max = e - ec
    assert eb_max % kdeg == 0 and eb_max % 128 == 0
    mesh = plsc.VectorSubcoreMesh(core_axis_name="c", subcore_axis_name="s")

    @functools.partial(
        pl.kernel,
        mesh=mesh,
        out_type=jax.ShapeDtypeStruct((npad, h), jnp.float32),
        scratch_types=[
            pltpu.VMEM((eblk, 128), jnp.int32),
            pltpu.VMEM((eblk, 128), jnp.int32),
            pltpu.VMEM((eblk, 128), jnp.int32),
            pltpu.VMEM((eblk, 128), jnp.int32),
            pltpu.VMEM((ec, h), jnp.float32),
            pltpu.VMEM((ec, h), jnp.float32),
            pltpu.VMEM((ec, h), jnp.float32),
            pltpu.VMEM((ec, h), jnp.float32),
            pltpu.VMEM((c_nodes, h), jnp.float32),
            pltpu.VMEM((c_nodes, h), jnp.float32),
            pltpu.VMEM_SHARED((n_tab, h), jnp.float32),
            pltpu.VMEM_SHARED((n_tab, h), jnp.float32),
            pltpu.SemaphoreType.DMA,
            pltpu.SemaphoreType.DMA,
            pltpu.SemaphoreType.DMA,
            pltpu.SemaphoreType.DMA,
            pltpu.SemaphoreType.DMA,
            pltpu.SemaphoreType.DMA,
        ],
        compiler_params=pltpu.CompilerParams(use_tc_tiling_on_sc=False,
                                             skip_device_barrier=True),
    )
    def run(adj_hbm, q_hbm, k_hbm, out_hbm,
            di0, si0, di1, si1, qr0, kr0, qr1, kr1, ob0, ob1, qs, ks,
            semi0, semi1, semg0, semg1, semo0, semo1):
        wid = lax.axis_index("s") * nc + lax.axis_index("c")
        ebase0 = wid * (np_w * kdeg)
        nbase0 = wid * np_w

        def eb_of(cix):
            raw = ebase0 + cix * ec
            return jnp.minimum(raw, eb_max), raw

        def start_idx(cix, di, si, sem):
            ebc, _ = eb_of(cix)
            bs = ebc // 128
            pltpu.async_copy(adj_hbm.at[pl.ds(bs, eblk), 1], di, sem)
            pltpu.async_copy(adj_hbm.at[pl.ds(bs, eblk), 0], si, sem)

        def wait_idx(di, si, sem):
            pltpu.make_async_copy(adj_hbm.at[pl.ds(0, eblk), 1], di, sem).wait()
            pltpu.make_async_copy(adj_hbm.at[pl.ds(0, eblk), 0], si, sem).wait()

        def start_gather(di, si, qr, kr, sem):
            for j in range(eblk):
                pltpu.async_copy(qs.at[di.at[j]],
                                 qr.at[pl.ds(j * 128, 128)], sem)
                pltpu.async_copy(ks.at[si.at[j]],
                                 kr.at[pl.ds(j * 128, 128)], sem)

        def wait_gather(di, si, qr, kr, sem):
            for j in range(eblk):
                pltpu.make_async_copy(qs.at[di.at[j]],
                                      qr.at[pl.ds(j * 128, 128)], sem).wait()
                pltpu.make_async_copy(ks.at[si.at[j]],
                                      kr.at[pl.ds(j * 128, 128)], sem).wait()

        def compute(cix, qr, kr, ob):
            ebc, raw = eb_of(cix)
            delta = raw - ebc

            def node_body(nix, carry):
                off = jnp.minimum(nix * kdeg + delta, ec - kdeg)
                acc = qr[off] * kr[off]
                for j in range(1, kdeg):
                    acc = acc + qr[off + j] * kr[off + j]
                ob[nix] = acc
                return carry

            lax.fori_loop(0, c_nodes, node_body, 0)

        def start_out(cix, ob, sem):
            pltpu.async_copy(
                ob, out_hbm.at[pl.ds(nbase0 + cix * c_nodes, c_nodes)], sem)

        def wait_out(ob, sem):
            pltpu.make_async_copy(
                ob, out_hbm.at[pl.ds(0, c_nodes)], sem).wait()

        start_idx(0, di0, si0, semi0)
        start_idx(1, di1, si1, semi1)
        sid = lax.axis_index("s")
        rows16 = n_tab // ns
        pltpu.sync_copy(q_hbm.at[pl.ds(sid * rows16, rows16)],
                        qs.at[pl.ds(sid * rows16, rows16)])
        pltpu.sync_copy(k_hbm.at[pl.ds(sid * rows16, rows16)],
                        ks.at[pl.ds(sid * rows16, rows16)])
        plsc.subcore_barrier()
        wait_idx(di0, si0, semi0)
        start_gather(di0, si0, qr0, kr0, semg0)

        def pair_body(t, carry):
            c0 = 2 * t
            c1 = c0 + 1
            wait_idx(di1, si1, semi1)
            start_gather(di1, si1, qr1, kr1, semg1)
            wait_gather(di0, si0, qr0, kr0, semg0)

            @pl.when(c0 + 2 < nchunk)
            def _():
                start_idx(c0 + 2, di0, si0, semi0)

            compute(c0, qr0, kr0, ob0)

            @pl.when(t > 0)
            def _():
                wait_out(ob0, semo0)

            start_out(c0, ob0, semo0)

            @pl.when(c0 + 2 < nchunk)
            def _():
                wait_idx(di0, si0, semi0)
                start_gather(di0, si0, qr0, kr0, semg0)

            wait_gather(di1, si1, qr1, kr1, semg1)

            @pl.when(c1 + 2 < nchunk)
            def _():
                start_idx(c1 + 2, di1, si1, semi1)

            compute(c1, qr1, kr1, ob1)

            @pl.when(t > 0)
            def _():
                wait_out(ob1, semo1)

            start_out(c1, ob1, semo1)
            return carry

        lax.fori_loop(0, nchunk // 2, pair_body, 0)
        wait_out(ob0, semo0)
        wait_out(ob1, semo1)

    return run(adj3, q_emb, k_s)


def _epi_body(q_ref, s_ref, we_ref, wv_ref, o_ref, *, h, pack):
    q = q_ref[...]
    we = _nonneg(we_ref[...])[0:1, :]
    s = we * (q * q) + s_ref[...]
    ri = lax.broadcasted_iota(jnp.int32, (128, 128), 0) // h
    ci = lax.broadcasted_iota(jnp.int32, (128, 128), 1) // h
    grp = (ri == ci).astype(jnp.float32)
    dn = (((1,), (0,)), ((), ()))
    norm = lax.dot_general(s, grp, dn, preferred_element_type=jnp.float32)
    attn = s / (norm + 1e-9)
    wv = _nonneg(wv_ref[...])
    dnt = (((1,), (1,)), ((), ()))
    for a in range(pack):
        o_ref[:, a, :] = lax.dot_general(
            attn[:, a * h:(a + 1) * h], wv, dnt,
            preferred_element_type=jnp.float32)


def _epilogue(q_pk, s_pk, we_tile, wv, n, npad, h, dout, pack, block_nodes):
    grid = npad // block_nodes
    bpk = block_nodes // pack
    return pl.pallas_call(
        functools.partial(_epi_body, h=h, pack=pack),
        grid=(grid,),
        in_specs=[
            pl.BlockSpec((bpk, 128), lambda i: (i, 0)),
            pl.BlockSpec((bpk, 128), lambda i: (i, 0)),
            pl.BlockSpec((8, 128), lambda i: (0, 0)),
            pl.BlockSpec((dout, h), lambda i: (0, 0)),
        ],
        out_specs=pl.BlockSpec((bpk, pack, dout), lambda i: (i, 0, 0)),
        out_shape=jax.ShapeDtypeStruct((n // pack, pack, dout), jnp.float32),
    )(q_pk, s_pk, we_tile, wv)


def kernel(adj_list, x, Wq, Wk, w_ego, Wv):
    n, d = x.shape
    e = adj_list.shape[1]
    h = Wq.shape[0]
    dout = Wv.shape[0]
    kdeg = e // n
    pack = 128 // h

    c_nodes = 16
    nw = 32
    npad = ((n + nw * c_nodes - 1) // (nw * c_nodes)) * (nw * c_nodes)

    adj3 = jnp.transpose(adj_list.reshape(2, e // 128, 128), (1, 0, 2))
    x3 = x.reshape(n // pack, pack, d)

    we_tile = jnp.tile(w_ego, (8, pack))

    q_pk, k_pk = _embeddings(x3, Wq, Wk, kdeg, npad, pack, block_nodes=2048)
    q_tab = q_pk.reshape(npad, h)
    k_tab = k_pk.reshape(npad, h)

    sum_local_pad = _sc_edge_sum(adj3, q_tab, k_tab,
                                 npad=npad, c_nodes=c_nodes, kdeg=kdeg)
    s_pk = sum_local_pad.reshape(npad // pack, 128)

    res3 = _epilogue(q_pk, s_pk, we_tile, Wv, n, npad, h, dout, pack,
                     block_nodes=2048)
    return res3.reshape(n, dout)
